# baseline (speedup 1.0000x reference)
import jax
import jax.numpy as jnp
from jax import lax
from jax.experimental import pallas as pl
from jax.experimental.pallas import tpu as pltpu
from jax.experimental.pallas import tpu_sc as plsc

_BATCH = 16384
_NF = 26
_FD = 38462
_EMB = 16

_NC = 2
_NS = 16
_NW = _NC * _NS

_TCOLS = 1000012
_TSTRIDE = 1000016

_ACHUNK = 1024
_A_FULL = _TCOLS // _ACHUNK
_A_TAIL = 512
_A_MAIN = 61
_TAIL0 = 999936
_NTAIL = 76

_NPAIR = _NF * _EMB
_PPW = _NPAIR // _NW
_L = 38472
_S0_MAX = _TSTRIDE - _L
_VECS = _BATCH // 16

_CCHUNK = 2048
_C_UNITS = _NF * 2 * (_BATCH // _CCHUNK)


def _detile_body(tT, tail_lin, tlin, tbuf, ltbuf, tailbuf, sem):
    wid = lax.axis_index("s") * _NC + lax.axis_index("c")

    def unit(u, carry):
        st = lax.rem(u, 2)
        k = u // 2
        row0 = pl.multiple_of(st * 8, 8)
        c0 = pl.multiple_of(k * _ACHUNK, 128)
        pltpu.sync_copy(tT.at[pl.ds(row0, 8), pl.ds(c0, _ACHUNK)], tbuf)
        for r in range(8):
            def cp(i, c2):
                ltbuf[pl.ds(r * _ACHUNK + i * 16, 16)] = tbuf[
                    r, pl.ds(i * 16, 16)
                ]
                return c2

            lax.fori_loop(0, _ACHUNK // 16, cp, 0, unroll=8)
        for r in range(8):
            pltpu.async_copy(
                ltbuf.at[pl.ds(r * _ACHUNK, _ACHUNK)],
                tlin.at[pl.ds((row0 + r) * _TSTRIDE + c0, _ACHUNK)],
                sem,
            )
        for r in range(8):
            pltpu.make_async_copy(
                ltbuf.at[pl.ds(r * _ACHUNK, _ACHUNK)],
                tlin.at[pl.ds((row0 + r) * _TSTRIDE + c0, _ACHUNK)],
                sem,
            ).wait()
        return carry

    def main(j, carry):
        return unit(wid + _NW * j, carry)

    lax.fori_loop(0, _A_MAIN, main, 0)

    @pl.when(wid < 2)
    def _():
        st = wid
        row0 = pl.multiple_of(st * 8, 8)
        c0 = _A_FULL * _ACHUNK
        pltpu.sync_copy(
            tT.at[pl.ds(row0, 8), pl.ds(c0, _A_TAIL)],
            tbuf.at[:, pl.ds(0, _A_TAIL)],
        )
        for r in range(8):
            def cp(i, c2):
                ltbuf[pl.ds(r * _A_TAIL + i * 16, 16)] = tbuf[
                    r, pl.ds(i * 16, 16)
                ]
                return c2

            lax.fori_loop(0, _A_TAIL // 16, cp, 0, unroll=8)
        for r in range(8):
            pltpu.async_copy(
                ltbuf.at[pl.ds(r * _A_TAIL, _A_TAIL)],
                tlin.at[pl.ds((row0 + r) * _TSTRIDE + c0, _A_TAIL)],
                sem,
            )
        for r in range(8):
            pltpu.make_async_copy(
                ltbuf.at[pl.ds(r * _A_TAIL, _A_TAIL)],
                tlin.at[pl.ds((row0 + r) * _TSTRIDE + c0, _A_TAIL)],
                sem,
            ).wait()

    @pl.when(wid == 2)
    def _():
        pltpu.sync_copy(tail_lin, tailbuf.at[pl.ds(0, _NTAIL * _EMB)])
        lane16 = lax.iota(jnp.int32, 16) * _EMB
        for e in range(_EMB):
            for k in range(5):
                iv = lane16 + (k * 16 * _EMB + e)
                ltbuf[pl.ds(k * 16, 16)] = plsc.load_gather(tailbuf, [iv])
            pltpu.sync_copy(
                ltbuf.at[pl.ds(0, 80)],
                tlin.at[pl.ds(e * _TSTRIDE + _TAIL0, 80)],
            )


def _gather_body(
    xf, tlin, olin, idx_v, s_a, s_b, o_a, o_b,
    sem_sa, sem_sb, sem_oa, sem_ob,
):
    wid = lax.axis_index("s") * _NC + lax.axis_index("c")
    q0 = wid * _PPW
    sbufs = (s_a, s_b)
    obufs = (o_a, o_b)
    ssems = (sem_sa, sem_sb)
    osems = (sem_oa, sem_ob)

    def params(j):
        q = q0 + j
        f = q // _EMB
        e = lax.rem(q, _EMB)
        off = f * _FD
        s0 = jnp.minimum((off // 8) * 8, _S0_MAX)
        return q, f, e, s0, off - s0

    q, f, e, s0, adj = params(0)
    pltpu.async_copy(tlin.at[pl.ds(e * _TSTRIDE + s0, _L)], sbufs[0], ssems[0])
    pltpu.sync_copy(xf.at[pl.ds(f * _BATCH, _BATCH)], idx_v)
    prev_f = f
    for j in range(_PPW):
        q, f, e, s0, adj = params(j)
        b = j % 2
        if j + 1 < _PPW:
            _, _, en, s0n, _ = params(j + 1)
            pltpu.async_copy(
                tlin.at[pl.ds(en * _TSTRIDE + s0n, _L)],
                sbufs[1 - b],
                ssems[1 - b],
            )

        @pl.when(f != prev_f)
        def _():
            pltpu.sync_copy(xf.at[pl.ds(f * _BATCH, _BATCH)], idx_v)

        prev_f = f
        pltpu.make_async_copy(
            tlin.at[pl.ds(e * _TSTRIDE + s0, _L)], sbufs[b], ssems[b]
        ).wait()
        if j >= 2:
            qp = q0 + j - 2
            pltpu.make_async_copy(
                obufs[b], olin.at[pl.ds(qp * _BATCH, _BATCH)], osems[b]
            ).wait()
        ob = obufs[b]
        sb = sbufs[b]

        def gb(i, carry):
            iv = idx_v[pl.ds(i * 16, 16)] + adj
            ob[pl.ds(i * 16, 16)] = plsc.load_gather(sb, [iv])
            return carry

        lax.fori_loop(0, _VECS, gb, 0, unroll=4)
        pltpu.async_copy(ob, olin.at[pl.ds(q * _BATCH, _BATCH)], osems[b])
    for j in (_PPW - 2, _PPW - 1):
        q = q0 + j
        b = j % 2
        pltpu.make_async_copy(
            obufs[b], olin.at[pl.ds(q * _BATCH, _BATCH)], osems[b]
        ).wait()


def _retile_body(
    olin, out, lo_a, lo_b, to_a, to_b, sem_la, sem_lb, sem_oa, sem_ob
):
    wid = lax.axis_index("s") * _NC + lax.axis_index("c")
    lobufs = (lo_a, lo_b)
    tobufs = (to_a, to_b)
    lsems = (sem_la, sem_lb)
    osems = (sem_oa, sem_ob)

    def params(j):
        u = wid * _PPW + j
        f = u // 16
        r2 = lax.rem(u, 16)
        e0 = pl.multiple_of((r2 // 8) * 8, 8)
        col0 = pl.multiple_of(lax.rem(r2, 8) * _CCHUNK, 128)
        return f, e0, col0

    def fetch(j, b):
        f, e0, col0 = params(j)
        for r in range(8):
            pltpu.async_copy(
                olin.at[pl.ds((f * _EMB + e0 + r) * _BATCH + col0, _CCHUNK)],
                lobufs[b].at[pl.ds(r * _CCHUNK, _CCHUNK)],
                lsems[b],
            )

    fetch(0, 0)
    for j in range(_PPW):
        f, e0, col0 = params(j)
        b = j % 2
        if j + 1 < _PPW:
            fetch(j + 1, 1 - b)
        for r in range(8):
            pltpu.make_async_copy(
                olin.at[pl.ds((f * _EMB + e0 + r) * _BATCH + col0, _CCHUNK)],
                lobufs[b].at[pl.ds(r * _CCHUNK, _CCHUNK)],
                lsems[b],
            ).wait()
        if j >= 2:
            fp, ep, cp0 = params(j - 2)
            pltpu.make_async_copy(
                tobufs[b],
                out.at[pl.ds(fp, 1), pl.ds(ep, 8), pl.ds(cp0, _CCHUNK)],
                osems[b],
            ).wait()
        lob = lobufs[b]
        tob = tobufs[b]
        for r in range(8):
            def cp(i, c2):
                tob[0, r, pl.ds(i * 16, 16)] = lob[
                    pl.ds(r * _CCHUNK + i * 16, 16)
                ]
                return c2

            lax.fori_loop(0, _CCHUNK // 16, cp, 0, unroll=8)
        pltpu.async_copy(
            tob,
            out.at[pl.ds(f, 1), pl.ds(e0, 8), pl.ds(col0, _CCHUNK)],
            osems[b],
        )
    for j in (_PPW - 2, _PPW - 1):
        f, e0, col0 = params(j)
        b = j % 2
        pltpu.make_async_copy(
            tobufs[b],
            out.at[pl.ds(f, 1), pl.ds(e0, 8), pl.ds(col0, _CCHUNK)],
            osems[b],
        ).wait()


_MESH = dict(core_axis_name="c", subcore_axis_name="s")
_CP = None


def kernel(x, table):
    cp = pltpu.CompilerParams(needs_layout_passes=False)
    mesh = plsc.VectorSubcoreMesh(**_MESH)

    tT = table.T
    xf = x.astype(jnp.int32).T.reshape(_NF * _BATCH)
    tail_lin = table[_TAIL0:_TCOLS, :].reshape(_NTAIL * _EMB)

    detile = pl.kernel(
        _detile_body,
        out_type=jax.ShapeDtypeStruct((_EMB * _TSTRIDE,), jnp.float32),
        mesh=mesh,
        scratch_types=[
            pltpu.VMEM((8, _ACHUNK), jnp.float32),
            pltpu.VMEM((8 * _ACHUNK,), jnp.float32),
            pltpu.VMEM((1280,), jnp.float32),
            pltpu.SemaphoreType.DMA,
        ],
        compiler_params=cp,
    )
    tlin = detile(tT, tail_lin)

    gather = pl.kernel(
        _gather_body,
        out_type=jax.ShapeDtypeStruct((_NPAIR * _BATCH,), jnp.float32),
        mesh=mesh,
        scratch_types=[
            pltpu.VMEM((_BATCH,), jnp.int32),
            pltpu.VMEM((_L,), jnp.float32),
            pltpu.VMEM((_L,), jnp.float32),
            pltpu.VMEM((_BATCH,), jnp.float32),
            pltpu.VMEM((_BATCH,), jnp.float32),
            pltpu.SemaphoreType.DMA,
            pltpu.SemaphoreType.DMA,
            pltpu.SemaphoreType.DMA,
            pltpu.SemaphoreType.DMA,
        ],
        compiler_params=cp,
    )
    olin = gather(xf, tlin)

    retile = pl.kernel(
        _retile_body,
        out_type=jax.ShapeDtypeStruct((_NF, _EMB, _BATCH), jnp.float32),
        mesh=mesh,
        scratch_types=[
            pltpu.VMEM((8 * _CCHUNK,), jnp.float32),
            pltpu.VMEM((8 * _CCHUNK,), jnp.float32),
            pltpu.VMEM((1, 8, _CCHUNK), jnp.float32),
            pltpu.VMEM((1, 8, _CCHUNK), jnp.float32),
            pltpu.SemaphoreType.DMA,
            pltpu.SemaphoreType.DMA,
            pltpu.SemaphoreType.DMA,
            pltpu.SemaphoreType.DMA,
        ],
        compiler_params=cp,
    )
    outT = retile(olin)
    return outT.transpose(2, 0, 1)


# speedup vs baseline: 1.5675x; 1.5675x over previous
import jax
import jax.numpy as jnp
from jax import lax
from jax.experimental import pallas as pl
from jax.experimental.pallas import tpu as pltpu
from jax.experimental.pallas import tpu_sc as plsc

_BATCH = 16384
_NF = 26
_FD = 38462
_EMB = 16

_NC = 2
_NS = 16
_NW = _NC * _NS

_TCOLS = 1000012
_TSTRIDE = 1000016

_ACHUNK = 1024
_A_FULL = _TCOLS // _ACHUNK
_A_TAIL = 512
_A_MAIN = 61
_TAIL0 = 999936
_NTAIL = 76

_NPAIR = _NF * _EMB
_PPW = _NPAIR // _NW
_L = 38472
_S0_MAX = _TSTRIDE - _L
_VECS = _BATCH // 16

_CCHUNK = 2048
_C_UNITS = _NF * 2 * (_BATCH // _CCHUNK)


def _detile_body(
    tT, tail_lin, tlin, tb_a, tb_b, lt_a, lt_b, tailbuf,
    sem_ia, sem_ib, sem_oa, sem_ob,
):
    wid = lax.axis_index("s") * _NC + lax.axis_index("c")
    tbufs = (tb_a, tb_b)
    ltbufs = (lt_a, lt_b)
    isems = (sem_ia, sem_ib)
    osems = (sem_oa, sem_ob)

    def uparams(k):
        u = wid + _NW * k
        row0 = pl.multiple_of(lax.rem(u, 2) * 8, 8)
        c0 = pl.multiple_of((u // 2) * _ACHUNK, 128)
        return row0, c0

    def fire_in(k, b):
        row0, c0 = uparams(k)
        pltpu.async_copy(
            tT.at[pl.ds(row0, 8), pl.ds(c0, _ACHUNK)], tbufs[b], isems[b]
        )

    def wait_in(k, b):
        row0, c0 = uparams(k)
        pltpu.make_async_copy(
            tT.at[pl.ds(row0, 8), pl.ds(c0, _ACHUNK)], tbufs[b], isems[b]
        ).wait()

    def fire_out(k, b):
        row0, c0 = uparams(k)
        for r in range(8):
            pltpu.async_copy(
                ltbufs[b].at[pl.ds(r * _ACHUNK, _ACHUNK)],
                tlin.at[pl.ds((row0 + r) * _TSTRIDE + c0, _ACHUNK)],
                osems[b],
            )

    def wait_out(k, b):
        row0, c0 = uparams(k)
        for r in range(8):
            pltpu.make_async_copy(
                ltbufs[b].at[pl.ds(r * _ACHUNK, _ACHUNK)],
                tlin.at[pl.ds((row0 + r) * _TSTRIDE + c0, _ACHUNK)],
                osems[b],
            ).wait()

    def compute(b):
        tb = tbufs[b]
        lt = ltbufs[b]
        for r in range(8):
            def cp(i, c2):
                lt[pl.ds(r * _ACHUNK + i * 16, 16)] = tb[r, pl.ds(i * 16, 16)]
                return c2

            lax.fori_loop(0, _ACHUNK // 16, cp, 0, unroll=8)

    fire_in(0, 0)

    def main(j, carry):
        k0 = 2 * j
        fire_in(k0 + 1, 1)
        wait_in(k0, 0)

        @pl.when(j > 0)
        def _():
            wait_out(k0 - 2, 0)

        compute(0)
        fire_out(k0, 0)
        fire_in(k0 + 2, 0)
        wait_in(k0 + 1, 1)

        @pl.when(j > 0)
        def _():
            wait_out(k0 - 1, 1)

        compute(1)
        fire_out(k0 + 1, 1)
        return carry

    lax.fori_loop(0, (_A_MAIN - 1) // 2, main, 0)
    wait_in(_A_MAIN - 1, 0)
    wait_out(_A_MAIN - 3, 0)
    compute(0)
    fire_out(_A_MAIN - 1, 0)
    wait_out(_A_MAIN - 2, 1)
    wait_out(_A_MAIN - 1, 0)

    @pl.when(wid < 2)
    def _():
        st = wid
        row0 = pl.multiple_of(st * 8, 8)
        c0 = _A_FULL * _ACHUNK
        pltpu.sync_copy(
            tT.at[pl.ds(row0, 8), pl.ds(c0, _A_TAIL)],
            tb_a.at[:, pl.ds(0, _A_TAIL)],
        )
        for r in range(8):
            def cp(i, c2):
                lt_a[pl.ds(r * _A_TAIL + i * 16, 16)] = tb_a[
                    r, pl.ds(i * 16, 16)
                ]
                return c2

            lax.fori_loop(0, _A_TAIL // 16, cp, 0, unroll=8)
        for r in range(8):
            pltpu.async_copy(
                lt_a.at[pl.ds(r * _A_TAIL, _A_TAIL)],
                tlin.at[pl.ds((row0 + r) * _TSTRIDE + c0, _A_TAIL)],
                sem_ia,
            )
        for r in range(8):
            pltpu.make_async_copy(
                lt_a.at[pl.ds(r * _A_TAIL, _A_TAIL)],
                tlin.at[pl.ds((row0 + r) * _TSTRIDE + c0, _A_TAIL)],
                sem_ia,
            ).wait()

    @pl.when(wid == 2)
    def _():
        pltpu.sync_copy(tail_lin, tailbuf.at[pl.ds(0, _NTAIL * _EMB)])
        lane16 = lax.iota(jnp.int32, 16) * _EMB
        for e in range(_EMB):
            for k in range(5):
                iv = lane16 + (k * 16 * _EMB + e)
                lt_a[pl.ds(k * 16, 16)] = plsc.load_gather(tailbuf, [iv])
            pltpu.sync_copy(
                lt_a.at[pl.ds(0, 80)],
                tlin.at[pl.ds(e * _TSTRIDE + _TAIL0, 80)],
            )


def _gather_body(
    xf, tlin, olin, idx_v, s_a, s_b, o_a, o_b,
    sem_sa, sem_sb, sem_oa, sem_ob,
):
    wid = lax.axis_index("s") * _NC + lax.axis_index("c")
    q0 = wid * _PPW
    sbufs = (s_a, s_b)
    obufs = (o_a, o_b)
    ssems = (sem_sa, sem_sb)
    osems = (sem_oa, sem_ob)

    def params(j):
        q = q0 + j
        f = q // _EMB
        e = lax.rem(q, _EMB)
        off = f * _FD
        s0 = jnp.minimum((off // 8) * 8, _S0_MAX)
        return q, f, e, s0, off - s0

    q, f, e, s0, adj = params(0)
    pltpu.async_copy(tlin.at[pl.ds(e * _TSTRIDE + s0, _L)], sbufs[0], ssems[0])
    pltpu.sync_copy(xf.at[pl.ds(f * _BATCH, _BATCH)], idx_v)
    prev_f = f
    for j in range(_PPW):
        q, f, e, s0, adj = params(j)
        b = j % 2
        if j + 1 < _PPW:
            _, _, en, s0n, _ = params(j + 1)
            pltpu.async_copy(
                tlin.at[pl.ds(en * _TSTRIDE + s0n, _L)],
                sbufs[1 - b],
                ssems[1 - b],
            )

        @pl.when(f != prev_f)
        def _():
            pltpu.sync_copy(xf.at[pl.ds(f * _BATCH, _BATCH)], idx_v)

        prev_f = f
        pltpu.make_async_copy(
            tlin.at[pl.ds(e * _TSTRIDE + s0, _L)], sbufs[b], ssems[b]
        ).wait()
        if j >= 2:
            qp = q0 + j - 2
            pltpu.make_async_copy(
                obufs[b], olin.at[pl.ds(qp * _BATCH, _BATCH)], osems[b]
            ).wait()
        ob = obufs[b]
        sb = sbufs[b]

        def gb(i, carry):
            iv = idx_v[pl.ds(i * 16, 16)] + adj
            ob[pl.ds(i * 16, 16)] = plsc.load_gather(sb, [iv])
            return carry

        lax.fori_loop(0, _VECS, gb, 0)
        pltpu.async_copy(ob, olin.at[pl.ds(q * _BATCH, _BATCH)], osems[b])
    for j in (_PPW - 2, _PPW - 1):
        q = q0 + j
        b = j % 2
        pltpu.make_async_copy(
            obufs[b], olin.at[pl.ds(q * _BATCH, _BATCH)], osems[b]
        ).wait()


def _retile_body(
    olin, out, lo_a, lo_b, to_a, to_b, sem_la, sem_lb, sem_oa, sem_ob
):
    wid = lax.axis_index("s") * _NC + lax.axis_index("c")
    lobufs = (lo_a, lo_b)
    tobufs = (to_a, to_b)
    lsems = (sem_la, sem_lb)
    osems = (sem_oa, sem_ob)

    def params(j):
        u = wid * _PPW + j
        f = u // 16
        r2 = lax.rem(u, 16)
        e0 = pl.multiple_of((r2 // 8) * 8, 8)
        col0 = pl.multiple_of(lax.rem(r2, 8) * _CCHUNK, 128)
        return f, e0, col0

    def fetch(j, b):
        f, e0, col0 = params(j)
        for r in range(8):
            pltpu.async_copy(
                olin.at[pl.ds((f * _EMB + e0 + r) * _BATCH + col0, _CCHUNK)],
                lobufs[b].at[pl.ds(r * _CCHUNK, _CCHUNK)],
                lsems[b],
            )

    fetch(0, 0)
    for j in range(_PPW):
        f, e0, col0 = params(j)
        b = j % 2
        if j + 1 < _PPW:
            fetch(j + 1, 1 - b)
        for r in range(8):
            pltpu.make_async_copy(
                olin.at[pl.ds((f * _EMB + e0 + r) * _BATCH + col0, _CCHUNK)],
                lobufs[b].at[pl.ds(r * _CCHUNK, _CCHUNK)],
                lsems[b],
            ).wait()
        if j >= 2:
            fp, ep, cp0 = params(j - 2)
            pltpu.make_async_copy(
                tobufs[b],
                out.at[pl.ds(fp, 1), pl.ds(ep, 8), pl.ds(cp0, _CCHUNK)],
                osems[b],
            ).wait()
        lob = lobufs[b]
        tob = tobufs[b]
        for r in range(8):
            def cp(i, c2):
                tob[0, r, pl.ds(i * 16, 16)] = lob[
                    pl.ds(r * _CCHUNK + i * 16, 16)
                ]
                return c2

            lax.fori_loop(0, _CCHUNK // 16, cp, 0, unroll=8)
        pltpu.async_copy(
            tob,
            out.at[pl.ds(f, 1), pl.ds(e0, 8), pl.ds(col0, _CCHUNK)],
            osems[b],
        )
    for j in (_PPW - 2, _PPW - 1):
        f, e0, col0 = params(j)
        b = j % 2
        pltpu.make_async_copy(
            tobufs[b],
            out.at[pl.ds(f, 1), pl.ds(e0, 8), pl.ds(col0, _CCHUNK)],
            osems[b],
        ).wait()


_MESH = dict(core_axis_name="c", subcore_axis_name="s")
_CP = None


def kernel(x, table):
    cp = pltpu.CompilerParams(needs_layout_passes=False)
    mesh = plsc.VectorSubcoreMesh(**_MESH)

    tT = table.T
    xf = x.astype(jnp.int32).T.reshape(_NF * _BATCH)
    tail_lin = table[_TAIL0:_TCOLS, :].reshape(_NTAIL * _EMB)

    detile = pl.kernel(
        _detile_body,
        out_type=jax.ShapeDtypeStruct((_EMB * _TSTRIDE,), jnp.float32),
        mesh=mesh,
        scratch_types=[
            pltpu.VMEM((8, _ACHUNK), jnp.float32),
            pltpu.VMEM((8, _ACHUNK), jnp.float32),
            pltpu.VMEM((8 * _ACHUNK,), jnp.float32),
            pltpu.VMEM((8 * _ACHUNK,), jnp.float32),
            pltpu.VMEM((1280,), jnp.float32),
            pltpu.SemaphoreType.DMA,
            pltpu.SemaphoreType.DMA,
            pltpu.SemaphoreType.DMA,
            pltpu.SemaphoreType.DMA,
        ],
        compiler_params=cp,
    )
    tlin = detile(tT, tail_lin)

    gather = pl.kernel(
        _gather_body,
        out_type=jax.ShapeDtypeStruct((_NPAIR * _BATCH,), jnp.float32),
        mesh=mesh,
        scratch_types=[
            pltpu.VMEM((_BATCH,), jnp.int32),
            pltpu.VMEM((_L,), jnp.float32),
            pltpu.VMEM((_L,), jnp.float32),
            pltpu.VMEM((_BATCH,), jnp.float32),
            pltpu.VMEM((_BATCH,), jnp.float32),
            pltpu.SemaphoreType.DMA,
            pltpu.SemaphoreType.DMA,
            pltpu.SemaphoreType.DMA,
            pltpu.SemaphoreType.DMA,
        ],
        compiler_params=cp,
    )
    olin = gather(xf, tlin)

    retile = pl.kernel(
        _retile_body,
        out_type=jax.ShapeDtypeStruct((_NF, _EMB, _BATCH), jnp.float32),
        mesh=mesh,
        scratch_types=[
            pltpu.VMEM((8 * _CCHUNK,), jnp.float32),
            pltpu.VMEM((8 * _CCHUNK,), jnp.float32),
            pltpu.VMEM((1, 8, _CCHUNK), jnp.float32),
            pltpu.VMEM((1, 8, _CCHUNK), jnp.float32),
            pltpu.SemaphoreType.DMA,
            pltpu.SemaphoreType.DMA,
            pltpu.SemaphoreType.DMA,
            pltpu.SemaphoreType.DMA,
        ],
        compiler_params=cp,
    )
    outT = retile(olin)
    return outT.transpose(2, 0, 1)


# speedup vs baseline: 2.2747x; 1.4511x over previous
import jax
import jax.numpy as jnp
from jax import lax
from jax.experimental import pallas as pl
from jax.experimental.pallas import tpu as pltpu
from jax.experimental.pallas import tpu_sc as plsc

_BATCH = 16384
_NF = 26
_FD = 38462
_EMB = 16

_NC = 2
_NS = 16
_NW = _NC * _NS

_TCOLS = 1000012
_TSTRIDE = 1000016

_ACHUNK = 1024
_A_FULL = _TCOLS // _ACHUNK
_A_TAIL = 512
_A_MAIN = 61
_TAIL0 = 999936
_NTAIL = 76

_NPAIR = _NF * _EMB
_PPW = _NPAIR // _NW
_L = 38472
_S0_MAX = _TSTRIDE - _L
_VECS = _BATCH // 16

_CCHUNK = 2048
_C_UNITS = _NF * 2 * (_BATCH // _CCHUNK)


def _detile_body(
    tT, tail_lin, tlin, tb_a, tb_b, lt_a, lt_b, tailbuf,
    sem_ia, sem_ib, sem_oa, sem_ob,
):
    wid = lax.axis_index("s") * _NC + lax.axis_index("c")
    tbufs = (tb_a, tb_b)
    ltbufs = (lt_a, lt_b)
    isems = (sem_ia, sem_ib)
    osems = (sem_oa, sem_ob)

    def uparams(k):
        u = wid + _NW * k
        row0 = pl.multiple_of(lax.rem(u, 2) * 8, 8)
        c0 = pl.multiple_of((u // 2) * _ACHUNK, 128)
        return row0, c0

    def fire_in(k, b):
        row0, c0 = uparams(k)
        pltpu.async_copy(
            tT.at[pl.ds(row0, 8), pl.ds(c0, _ACHUNK)], tbufs[b], isems[b]
        )

    def wait_in(k, b):
        row0, c0 = uparams(k)
        pltpu.make_async_copy(
            tT.at[pl.ds(row0, 8), pl.ds(c0, _ACHUNK)], tbufs[b], isems[b]
        ).wait()

    def fire_out(k, b):
        row0, c0 = uparams(k)
        for r in range(8):
            pltpu.async_copy(
                ltbufs[b].at[pl.ds(r * _ACHUNK, _ACHUNK)],
                tlin.at[pl.ds((row0 + r) * _TSTRIDE + c0, _ACHUNK)],
                osems[b],
            )

    def wait_out(k, b):
        row0, c0 = uparams(k)
        for r in range(8):
            pltpu.make_async_copy(
                ltbufs[b].at[pl.ds(r * _ACHUNK, _ACHUNK)],
                tlin.at[pl.ds((row0 + r) * _TSTRIDE + c0, _ACHUNK)],
                osems[b],
            ).wait()

    def compute(b):
        tb = tbufs[b]
        lt = ltbufs[b]
        for r in range(8):
            def cp(i, c2):
                lt[pl.ds(r * _ACHUNK + i * 16, 16)] = tb[r, pl.ds(i * 16, 16)]
                return c2

            lax.fori_loop(0, _ACHUNK // 16, cp, 0, unroll=8)

    fire_in(0, 0)

    def main(j, carry):
        k0 = 2 * j
        fire_in(k0 + 1, 1)
        wait_in(k0, 0)

        @pl.when(j > 0)
        def _():
            wait_out(k0 - 2, 0)

        compute(0)
        fire_out(k0, 0)
        fire_in(k0 + 2, 0)
        wait_in(k0 + 1, 1)

        @pl.when(j > 0)
        def _():
            wait_out(k0 - 1, 1)

        compute(1)
        fire_out(k0 + 1, 1)
        return carry

    lax.fori_loop(0, (_A_MAIN - 1) // 2, main, 0)
    wait_in(_A_MAIN - 1, 0)
    wait_out(_A_MAIN - 3, 0)
    compute(0)
    fire_out(_A_MAIN - 1, 0)
    wait_out(_A_MAIN - 2, 1)
    wait_out(_A_MAIN - 1, 0)

    @pl.when(wid < 2)
    def _():
        st = wid
        row0 = pl.multiple_of(st * 8, 8)
        c0 = _A_FULL * _ACHUNK
        pltpu.sync_copy(
            tT.at[pl.ds(row0, 8), pl.ds(c0, _A_TAIL)],
            tb_a.at[:, pl.ds(0, _A_TAIL)],
        )
        for r in range(8):
            def cp(i, c2):
                lt_a[pl.ds(r * _A_TAIL + i * 16, 16)] = tb_a[
                    r, pl.ds(i * 16, 16)
                ]
                return c2

            lax.fori_loop(0, _A_TAIL // 16, cp, 0, unroll=8)
        for r in range(8):
            pltpu.async_copy(
                lt_a.at[pl.ds(r * _A_TAIL, _A_TAIL)],
                tlin.at[pl.ds((row0 + r) * _TSTRIDE + c0, _A_TAIL)],
                sem_ia,
            )
        for r in range(8):
            pltpu.make_async_copy(
                lt_a.at[pl.ds(r * _A_TAIL, _A_TAIL)],
                tlin.at[pl.ds((row0 + r) * _TSTRIDE + c0, _A_TAIL)],
                sem_ia,
            ).wait()

    @pl.when(wid == 2)
    def _():
        pltpu.sync_copy(tail_lin, tailbuf.at[pl.ds(0, _NTAIL * _EMB)])
        lane16 = lax.iota(jnp.int32, 16) * _EMB
        for e in range(_EMB):
            for k in range(5):
                iv = lane16 + (k * 16 * _EMB + e)
                lt_a[pl.ds(k * 16, 16)] = plsc.load_gather(tailbuf, [iv])
            pltpu.sync_copy(
                lt_a.at[pl.ds(0, 80)],
                tlin.at[pl.ds(e * _TSTRIDE + _TAIL0, 80)],
            )


def _gather_body(
    xf, tlin, olin, idx_v, s_a, s_b, o_a, o_b,
    sem_sa, sem_sb, sem_oa, sem_ob,
):
    wid = lax.axis_index("s") * _NC + lax.axis_index("c")
    q0 = wid * _PPW
    sbufs = (s_a, s_b)
    obufs = (o_a, o_b)
    ssems = (sem_sa, sem_sb)
    osems = (sem_oa, sem_ob)

    def params(j):
        q = q0 + j
        f = q // _EMB
        e = lax.rem(q, _EMB)
        off = f * _FD
        s0 = jnp.minimum((off // 8) * 8, _S0_MAX)
        return q, f, e, s0, off - s0

    q, f, e, s0, adj = params(0)
    pltpu.async_copy(tlin.at[pl.ds(e * _TSTRIDE + s0, _L)], sbufs[0], ssems[0])
    pltpu.sync_copy(xf.at[pl.ds(f * _BATCH, _BATCH)], idx_v)
    prev_f = f
    for j in range(_PPW):
        q, f, e, s0, adj = params(j)
        b = j % 2
        if j + 1 < _PPW:
            _, _, en, s0n, _ = params(j + 1)
            pltpu.async_copy(
                tlin.at[pl.ds(en * _TSTRIDE + s0n, _L)],
                sbufs[1 - b],
                ssems[1 - b],
            )

        @pl.when(f != prev_f)
        def _():
            pltpu.sync_copy(xf.at[pl.ds(f * _BATCH, _BATCH)], idx_v)

        prev_f = f
        pltpu.make_async_copy(
            tlin.at[pl.ds(e * _TSTRIDE + s0, _L)], sbufs[b], ssems[b]
        ).wait()
        if j >= 2:
            qp = q0 + j - 2
            pltpu.make_async_copy(
                obufs[b], olin.at[pl.ds(qp * _BATCH, _BATCH)], osems[b]
            ).wait()
        ob = obufs[b]
        sb = sbufs[b]

        @plsc.parallel_loop(0, _VECS, unroll=4)
        def _(i):
            iv = idx_v[pl.ds(i * 16, 16)] + adj
            ob[pl.ds(i * 16, 16)] = plsc.load_gather(sb, [iv])
        pltpu.async_copy(ob, olin.at[pl.ds(q * _BATCH, _BATCH)], osems[b])
    for j in (_PPW - 2, _PPW - 1):
        q = q0 + j
        b = j % 2
        pltpu.make_async_copy(
            obufs[b], olin.at[pl.ds(q * _BATCH, _BATCH)], osems[b]
        ).wait()


def _retile_body(
    olin, out, lo_a, lo_b, to_a, to_b, sem_la, sem_lb, sem_oa, sem_ob
):
    wid = lax.axis_index("s") * _NC + lax.axis_index("c")
    lobufs = (lo_a, lo_b)
    tobufs = (to_a, to_b)
    lsems = (sem_la, sem_lb)
    osems = (sem_oa, sem_ob)

    def params(j):
        u = wid * _PPW + j
        f = u // 16
        r2 = lax.rem(u, 16)
        e0 = pl.multiple_of((r2 // 8) * 8, 8)
        col0 = pl.multiple_of(lax.rem(r2, 8) * _CCHUNK, 128)
        return f, e0, col0

    def fetch(j, b):
        f, e0, col0 = params(j)
        for r in range(8):
            pltpu.async_copy(
                olin.at[pl.ds((f * _EMB + e0 + r) * _BATCH + col0, _CCHUNK)],
                lobufs[b].at[pl.ds(r * _CCHUNK, _CCHUNK)],
                lsems[b],
            )

    fetch(0, 0)
    for j in range(_PPW):
        f, e0, col0 = params(j)
        b = j % 2
        if j + 1 < _PPW:
            fetch(j + 1, 1 - b)
        for r in range(8):
            pltpu.make_async_copy(
                olin.at[pl.ds((f * _EMB + e0 + r) * _BATCH + col0, _CCHUNK)],
                lobufs[b].at[pl.ds(r * _CCHUNK, _CCHUNK)],
                lsems[b],
            ).wait()
        if j >= 2:
            fp, ep, cp0 = params(j - 2)
            pltpu.make_async_copy(
                tobufs[b],
                out.at[pl.ds(fp, 1), pl.ds(ep, 8), pl.ds(cp0, _CCHUNK)],
                osems[b],
            ).wait()
        lob = lobufs[b]
        tob = tobufs[b]
        for r in range(8):
            @plsc.parallel_loop(0, _CCHUNK // 16, unroll=8)
            def _(i, r=r):
                tob[0, r, pl.ds(i * 16, 16)] = lob[
                    pl.ds(r * _CCHUNK + i * 16, 16)
                ]
        pltpu.async_copy(
            tob,
            out.at[pl.ds(f, 1), pl.ds(e0, 8), pl.ds(col0, _CCHUNK)],
            osems[b],
        )
    for j in (_PPW - 2, _PPW - 1):
        f, e0, col0 = params(j)
        b = j % 2
        pltpu.make_async_copy(
            tobufs[b],
            out.at[pl.ds(f, 1), pl.ds(e0, 8), pl.ds(col0, _CCHUNK)],
            osems[b],
        ).wait()


_MESH = dict(core_axis_name="c", subcore_axis_name="s")
_CP = None


def kernel(x, table):
    cp = pltpu.CompilerParams(needs_layout_passes=False)
    mesh = plsc.VectorSubcoreMesh(**_MESH)

    tT = table.T
    xf = x.astype(jnp.int32).T.reshape(_NF * _BATCH)
    tail_lin = table[_TAIL0:_TCOLS, :].reshape(_NTAIL * _EMB)

    detile = pl.kernel(
        _detile_body,
        out_type=jax.ShapeDtypeStruct((_EMB * _TSTRIDE,), jnp.float32),
        mesh=mesh,
        scratch_types=[
            pltpu.VMEM((8, _ACHUNK), jnp.float32),
            pltpu.VMEM((8, _ACHUNK), jnp.float32),
            pltpu.VMEM((8 * _ACHUNK,), jnp.float32),
            pltpu.VMEM((8 * _ACHUNK,), jnp.float32),
            pltpu.VMEM((1280,), jnp.float32),
            pltpu.SemaphoreType.DMA,
            pltpu.SemaphoreType.DMA,
            pltpu.SemaphoreType.DMA,
            pltpu.SemaphoreType.DMA,
        ],
        compiler_params=cp,
    )
    tlin = detile(tT, tail_lin)

    gather = pl.kernel(
        _gather_body,
        out_type=jax.ShapeDtypeStruct((_NPAIR * _BATCH,), jnp.float32),
        mesh=mesh,
        scratch_types=[
            pltpu.VMEM((_BATCH,), jnp.int32),
            pltpu.VMEM((_L,), jnp.float32),
            pltpu.VMEM((_L,), jnp.float32),
            pltpu.VMEM((_BATCH,), jnp.float32),
            pltpu.VMEM((_BATCH,), jnp.float32),
            pltpu.SemaphoreType.DMA,
            pltpu.SemaphoreType.DMA,
            pltpu.SemaphoreType.DMA,
            pltpu.SemaphoreType.DMA,
        ],
        compiler_params=cp,
    )
    olin = gather(xf, tlin)

    retile = pl.kernel(
        _retile_body,
        out_type=jax.ShapeDtypeStruct((_NF, _EMB, _BATCH), jnp.float32),
        mesh=mesh,
        scratch_types=[
            pltpu.VMEM((8 * _CCHUNK,), jnp.float32),
            pltpu.VMEM((8 * _CCHUNK,), jnp.float32),
            pltpu.VMEM((1, 8, _CCHUNK), jnp.float32),
            pltpu.VMEM((1, 8, _CCHUNK), jnp.float32),
            pltpu.SemaphoreType.DMA,
            pltpu.SemaphoreType.DMA,
            pltpu.SemaphoreType.DMA,
            pltpu.SemaphoreType.DMA,
        ],
        compiler_params=cp,
    )
    outT = retile(olin)
    return outT.transpose(2, 0, 1)


# speedup vs baseline: 2.2965x; 1.0096x over previous
import jax
import jax.numpy as jnp
from jax import lax
from jax.experimental import pallas as pl
from jax.experimental.pallas import tpu as pltpu
from jax.experimental.pallas import tpu_sc as plsc

_BATCH = 16384
_NF = 26
_FD = 38462
_EMB = 16

_NC = 2
_NS = 16
_NW = _NC * _NS

_TCOLS = 1000012
_TSTRIDE = 1000016

_ACHUNK = 1024
_A_FULL = _TCOLS // _ACHUNK
_A_TAIL = 512
_A_MAIN = 61
_TAIL0 = 999936
_NTAIL = 76

_NPAIR = _NF * _EMB
_PPW = _NPAIR // _NW
_L = 38472
_S0_MAX = _TSTRIDE - _L
_VECS = _BATCH // 16

_CCHUNK = 2048
_C_UNITS = _NF * 2 * (_BATCH // _CCHUNK)


def _detile_body(
    tT, tail_lin, tlin, tb_a, tb_b, lt_a, lt_b, tailbuf,
    sem_ia, sem_ib, sem_oa, sem_ob,
):
    wid = lax.axis_index("s") * _NC + lax.axis_index("c")
    tbufs = (tb_a, tb_b)
    ltbufs = (lt_a, lt_b)
    isems = (sem_ia, sem_ib)
    osems = (sem_oa, sem_ob)

    def uparams(k):
        u = wid + _NW * k
        row0 = pl.multiple_of(lax.rem(u, 2) * 8, 8)
        c0 = pl.multiple_of((u // 2) * _ACHUNK, 128)
        return row0, c0

    def fire_in(k, b):
        row0, c0 = uparams(k)
        pltpu.async_copy(
            tT.at[pl.ds(row0, 8), pl.ds(c0, _ACHUNK)], tbufs[b], isems[b]
        )

    def wait_in(k, b):
        row0, c0 = uparams(k)
        pltpu.make_async_copy(
            tT.at[pl.ds(row0, 8), pl.ds(c0, _ACHUNK)], tbufs[b], isems[b]
        ).wait()

    def fire_out(k, b):
        row0, c0 = uparams(k)
        for r in range(8):
            pltpu.async_copy(
                ltbufs[b].at[pl.ds(r * _ACHUNK, _ACHUNK)],
                tlin.at[pl.ds((row0 + r) * _TSTRIDE + c0, _ACHUNK)],
                osems[b],
            )

    def wait_out(k, b):
        row0, c0 = uparams(k)
        for r in range(8):
            pltpu.make_async_copy(
                ltbufs[b].at[pl.ds(r * _ACHUNK, _ACHUNK)],
                tlin.at[pl.ds((row0 + r) * _TSTRIDE + c0, _ACHUNK)],
                osems[b],
            ).wait()

    def compute(b):
        tb = tbufs[b]
        lt = ltbufs[b]
        for r in range(8):
            @plsc.parallel_loop(0, _ACHUNK // 16, unroll=8)
            def _(i, r=r):
                lt[pl.ds(r * _ACHUNK + i * 16, 16)] = tb[r, pl.ds(i * 16, 16)]

    fire_in(0, 0)

    def main(j, carry):
        k0 = 2 * j
        fire_in(k0 + 1, 1)
        wait_in(k0, 0)

        @pl.when(j > 0)
        def _():
            wait_out(k0 - 2, 0)

        compute(0)
        fire_out(k0, 0)
        fire_in(k0 + 2, 0)
        wait_in(k0 + 1, 1)

        @pl.when(j > 0)
        def _():
            wait_out(k0 - 1, 1)

        compute(1)
        fire_out(k0 + 1, 1)
        return carry

    lax.fori_loop(0, (_A_MAIN - 1) // 2, main, 0)
    wait_in(_A_MAIN - 1, 0)
    wait_out(_A_MAIN - 3, 0)
    compute(0)
    fire_out(_A_MAIN - 1, 0)
    wait_out(_A_MAIN - 2, 1)
    wait_out(_A_MAIN - 1, 0)

    @pl.when(wid < 2)
    def _():
        st = wid
        row0 = pl.multiple_of(st * 8, 8)
        c0 = _A_FULL * _ACHUNK
        pltpu.sync_copy(
            tT.at[pl.ds(row0, 8), pl.ds(c0, _A_TAIL)],
            tb_a.at[:, pl.ds(0, _A_TAIL)],
        )
        for r in range(8):
            @plsc.parallel_loop(0, _A_TAIL // 16, unroll=8)
            def _(i, r=r):
                lt_a[pl.ds(r * _A_TAIL + i * 16, 16)] = tb_a[
                    r, pl.ds(i * 16, 16)
                ]
        for r in range(8):
            pltpu.async_copy(
                lt_a.at[pl.ds(r * _A_TAIL, _A_TAIL)],
                tlin.at[pl.ds((row0 + r) * _TSTRIDE + c0, _A_TAIL)],
                sem_ia,
            )
        for r in range(8):
            pltpu.make_async_copy(
                lt_a.at[pl.ds(r * _A_TAIL, _A_TAIL)],
                tlin.at[pl.ds((row0 + r) * _TSTRIDE + c0, _A_TAIL)],
                sem_ia,
            ).wait()

    @pl.when(wid == 2)
    def _():
        pltpu.sync_copy(tail_lin, tailbuf.at[pl.ds(0, _NTAIL * _EMB)])
        lane16 = lax.iota(jnp.int32, 16) * _EMB
        for e in range(_EMB):
            for k in range(5):
                iv = lane16 + (k * 16 * _EMB + e)
                lt_a[pl.ds(k * 16, 16)] = plsc.load_gather(tailbuf, [iv])
            pltpu.sync_copy(
                lt_a.at[pl.ds(0, 80)],
                tlin.at[pl.ds(e * _TSTRIDE + _TAIL0, 80)],
            )


def _gather_body(
    xf, tlin, olin, idx_v, s_a, s_b, o_a, o_b,
    sem_sa, sem_sb, sem_oa, sem_ob,
):
    wid = lax.axis_index("s") * _NC + lax.axis_index("c")
    q0 = wid * _PPW
    sbufs = (s_a, s_b)
    obufs = (o_a, o_b)
    ssems = (sem_sa, sem_sb)
    osems = (sem_oa, sem_ob)

    def params(j):
        q = q0 + j
        f = q // _EMB
        e = lax.rem(q, _EMB)
        off = f * _FD
        s0 = jnp.minimum((off // 8) * 8, _S0_MAX)
        return q, f, e, s0, off - s0

    q, f, e, s0, adj = params(0)
    pltpu.async_copy(tlin.at[pl.ds(e * _TSTRIDE + s0, _L)], sbufs[0], ssems[0])
    pltpu.sync_copy(xf.at[pl.ds(f * _BATCH, _BATCH)], idx_v)
    prev_f = f
    for j in range(_PPW):
        q, f, e, s0, adj = params(j)
        b = j % 2
        if j + 1 < _PPW:
            _, _, en, s0n, _ = params(j + 1)
            pltpu.async_copy(
                tlin.at[pl.ds(en * _TSTRIDE + s0n, _L)],
                sbufs[1 - b],
                ssems[1 - b],
            )

        @pl.when(f != prev_f)
        def _():
            pltpu.sync_copy(xf.at[pl.ds(f * _BATCH, _BATCH)], idx_v)

        prev_f = f
        pltpu.make_async_copy(
            tlin.at[pl.ds(e * _TSTRIDE + s0, _L)], sbufs[b], ssems[b]
        ).wait()
        if j >= 2:
            qp = q0 + j - 2
            pltpu.make_async_copy(
                obufs[b], olin.at[pl.ds(qp * _BATCH, _BATCH)], osems[b]
            ).wait()
        ob = obufs[b]
        sb = sbufs[b]

        @plsc.parallel_loop(0, _VECS, unroll=4)
        def _(i):
            iv = idx_v[pl.ds(i * 16, 16)] + adj
            ob[pl.ds(i * 16, 16)] = plsc.load_gather(sb, [iv])
        pltpu.async_copy(ob, olin.at[pl.ds(q * _BATCH, _BATCH)], osems[b])
    for j in (_PPW - 2, _PPW - 1):
        q = q0 + j
        b = j % 2
        pltpu.make_async_copy(
            obufs[b], olin.at[pl.ds(q * _BATCH, _BATCH)], osems[b]
        ).wait()


def _retile_body(
    olin, out, lo_a, lo_b, to_a, to_b, sem_la, sem_lb, sem_oa, sem_ob
):
    wid = lax.axis_index("s") * _NC + lax.axis_index("c")
    lobufs = (lo_a, lo_b)
    tobufs = (to_a, to_b)
    lsems = (sem_la, sem_lb)
    osems = (sem_oa, sem_ob)

    def params(j):
        u = wid * _PPW + j
        f = u // 16
        r2 = lax.rem(u, 16)
        e0 = pl.multiple_of((r2 // 8) * 8, 8)
        col0 = pl.multiple_of(lax.rem(r2, 8) * _CCHUNK, 128)
        return f, e0, col0

    def fetch(j, b):
        f, e0, col0 = params(j)
        for r in range(8):
            pltpu.async_copy(
                olin.at[pl.ds((f * _EMB + e0 + r) * _BATCH + col0, _CCHUNK)],
                lobufs[b].at[pl.ds(r * _CCHUNK, _CCHUNK)],
                lsems[b],
            )

    fetch(0, 0)
    for j in range(_PPW):
        f, e0, col0 = params(j)
        b = j % 2
        if j + 1 < _PPW:
            fetch(j + 1, 1 - b)
        for r in range(8):
            pltpu.make_async_copy(
                olin.at[pl.ds((f * _EMB + e0 + r) * _BATCH + col0, _CCHUNK)],
                lobufs[b].at[pl.ds(r * _CCHUNK, _CCHUNK)],
                lsems[b],
            ).wait()
        if j >= 2:
            fp, ep, cp0 = params(j - 2)
            pltpu.make_async_copy(
                tobufs[b],
                out.at[pl.ds(fp, 1), pl.ds(ep, 8), pl.ds(cp0, _CCHUNK)],
                osems[b],
            ).wait()
        lob = lobufs[b]
        tob = tobufs[b]
        for r in range(8):
            @plsc.parallel_loop(0, _CCHUNK // 16, unroll=8)
            def _(i, r=r):
                tob[0, r, pl.ds(i * 16, 16)] = lob[
                    pl.ds(r * _CCHUNK + i * 16, 16)
                ]
        pltpu.async_copy(
            tob,
            out.at[pl.ds(f, 1), pl.ds(e0, 8), pl.ds(col0, _CCHUNK)],
            osems[b],
        )
    for j in (_PPW - 2, _PPW - 1):
        f, e0, col0 = params(j)
        b = j % 2
        pltpu.make_async_copy(
            tobufs[b],
            out.at[pl.ds(f, 1), pl.ds(e0, 8), pl.ds(col0, _CCHUNK)],
            osems[b],
        ).wait()


_MESH = dict(core_axis_name="c", subcore_axis_name="s")
_CP = None


def kernel(x, table):
    cp = pltpu.CompilerParams(needs_layout_passes=False)
    mesh = plsc.VectorSubcoreMesh(**_MESH)

    tT = table.T
    xf = x.astype(jnp.int32).T.reshape(_NF * _BATCH)
    tail_lin = table[_TAIL0:_TCOLS, :].reshape(_NTAIL * _EMB)

    detile = pl.kernel(
        _detile_body,
        out_type=jax.ShapeDtypeStruct((_EMB * _TSTRIDE,), jnp.float32),
        mesh=mesh,
        scratch_types=[
            pltpu.VMEM((8, _ACHUNK), jnp.float32),
            pltpu.VMEM((8, _ACHUNK), jnp.float32),
            pltpu.VMEM((8 * _ACHUNK,), jnp.float32),
            pltpu.VMEM((8 * _ACHUNK,), jnp.float32),
            pltpu.VMEM((1280,), jnp.float32),
            pltpu.SemaphoreType.DMA,
            pltpu.SemaphoreType.DMA,
            pltpu.SemaphoreType.DMA,
            pltpu.SemaphoreType.DMA,
        ],
        compiler_params=cp,
    )
    tlin = detile(tT, tail_lin)

    gather = pl.kernel(
        _gather_body,
        out_type=jax.ShapeDtypeStruct((_NPAIR * _BATCH,), jnp.float32),
        mesh=mesh,
        scratch_types=[
            pltpu.VMEM((_BATCH,), jnp.int32),
            pltpu.VMEM((_L,), jnp.float32),
            pltpu.VMEM((_L,), jnp.float32),
            pltpu.VMEM((_BATCH,), jnp.float32),
            pltpu.VMEM((_BATCH,), jnp.float32),
            pltpu.SemaphoreType.DMA,
            pltpu.SemaphoreType.DMA,
            pltpu.SemaphoreType.DMA,
            pltpu.SemaphoreType.DMA,
        ],
        compiler_params=cp,
    )
    olin = gather(xf, tlin)

    retile = pl.kernel(
        _retile_body,
        out_type=jax.ShapeDtypeStruct((_NF, _EMB, _BATCH), jnp.float32),
        mesh=mesh,
        scratch_types=[
            pltpu.VMEM((8 * _CCHUNK,), jnp.float32),
            pltpu.VMEM((8 * _CCHUNK,), jnp.float32),
            pltpu.VMEM((1, 8, _CCHUNK), jnp.float32),
            pltpu.VMEM((1, 8, _CCHUNK), jnp.float32),
            pltpu.SemaphoreType.DMA,
            pltpu.SemaphoreType.DMA,
            pltpu.SemaphoreType.DMA,
            pltpu.SemaphoreType.DMA,
        ],
        compiler_params=cp,
    )
    outT = retile(olin)
    return outT.transpose(2, 0, 1)
